# baseline (speedup 1.0000x reference)
import functools

import jax
import jax.numpy as jnp
from jax import lax
from jax.experimental import pallas as pl
from jax.experimental.pallas import tpu as pltpu
from jax.experimental.pallas import tpu_sc as plsc

_VOCAB = 1000000
_D = 64
_ENT_V = 100000
_ENT_D = 100
_B = 16384
_L = 20

_NC = 2
_NS = 16
_NW = _NC * _NS
_POS = _B * _L
_PER_W = _POS // _NW
_CHUNK = 128
_NCHUNK = _PER_W // _CHUNK
_NBUF = 4

_BLK = 4096
_M_W = 503808
_SB_W = 122
_M_E = 53248
_SB_E = 12
_FIN_NB = 512


def _transform_body(xa_ref, xb_ref, w_ref, b_ref, out_ref):
    dn = (((0,), (0,)), ((), ()))
    acc_a = lax.dot_general(xa_ref[...], w_ref[...], dn,
                            preferred_element_type=jnp.float32)
    acc_b = lax.dot_general(xb_ref[...], w_ref[...], dn,
                            preferred_element_type=jnp.float32)
    bias = b_ref[...]
    out_ref[...] = jnp.tanh(
        jnp.concatenate([acc_a + bias, acc_b + bias], axis=1))


def _transform_table(entity_table_t, W, b):
    nblk = _M_E // _BLK
    return pl.pallas_call(
        _transform_body,
        grid=(nblk,),
        in_specs=[
            pl.BlockSpec((_ENT_D, _BLK), lambda i: (0, i)),
            pl.BlockSpec((_ENT_D, _BLK), lambda i: (0, _SB_E + i)),
            pl.BlockSpec((_ENT_D, _D), lambda i: (0, 0)),
            pl.BlockSpec((1, _D), lambda i: (0, 0)),
        ],
        out_specs=pl.BlockSpec((_BLK, 2 * _D), lambda i: (i, 0)),
        out_shape=jax.ShapeDtypeStruct((_M_E, 2 * _D), jnp.float32),
    )(entity_table_t, entity_table_t, W, b.reshape(1, _D))


def _wt_body(xa_ref, xb_ref, out_ref):
    i0 = lax.broadcasted_iota(jnp.int32, (_D, _D), 0)
    i1 = lax.broadcasted_iota(jnp.int32, (_D, _D), 1)
    eye = jnp.where(i0 == i1, 1.0, 0.0).astype(jnp.float32)
    dn = (((0,), (0,)), ((), ()))
    ya = lax.dot_general(xa_ref[...], eye, dn,
                         preferred_element_type=jnp.float32)
    yb = lax.dot_general(xb_ref[...], eye, dn,
                         preferred_element_type=jnp.float32)
    out_ref[...] = jnp.concatenate([ya, yb], axis=1)


def _transpose_word(word_table_t):
    nblk = _M_W // _BLK
    return pl.pallas_call(
        _wt_body,
        grid=(nblk,),
        in_specs=[
            pl.BlockSpec((_D, _BLK), lambda i: (0, i)),
            pl.BlockSpec((_D, _BLK), lambda i: (0, _SB_W + i)),
        ],
        out_specs=pl.BlockSpec((_BLK, 2 * _D), lambda i: (i, 0)),
        out_shape=jax.ShapeDtypeStruct((_M_W, 2 * _D), jnp.float32),
    )(word_table_t, word_table_t)


def _sc_body(table_hbm, src_hbm, out_hbm, src_v,
             buf0, buf1, buf2, buf3, sem0, sem1, sem2, sem3):
    c = lax.axis_index("c")
    s = lax.axis_index("s")
    wid = s * _NC + c
    bufs = (buf0, buf1, buf2, buf3)
    sems = (sem0, sem1, sem2, sem3)
    row0 = wid * _PER_W

    pltpu.sync_copy(src_hbm.at[wid], src_v)

    for t in range(_NBUF):
        pltpu.async_copy(table_hbm.at[src_v.at[t]], bufs[t], sems[t])

    def outer(o, carry):
        for t in range(_NBUF):
            j = o * _NBUF + t
            pltpu.make_async_copy(
                table_hbm.at[src_v.at[j]], bufs[t], sems[t]).wait()
            pltpu.async_copy(
                bufs[t], out_hbm.at[pl.ds(row0 + j * _CHUNK, _CHUNK)],
                sems[t]).wait()
            pltpu.async_copy(
                table_hbm.at[src_v.at[j + _NBUF]], bufs[t], sems[t])
        return carry

    lax.fori_loop(0, _NCHUNK // _NBUF - 1, outer, 0)

    for t in range(_NBUF):
        j = (_NCHUNK - _NBUF) + t
        pltpu.make_async_copy(
            table_hbm.at[src_v.at[j]], bufs[t], sems[t]).wait()
        pltpu.async_copy(
            bufs[t], out_hbm.at[pl.ds(row0 + j * _CHUNK, _CHUNK)],
            sems[t]).wait()


@functools.partial(
    pl.kernel,
    out_type=jax.ShapeDtypeStruct((_POS, _D), jnp.float32),
    mesh=plsc.VectorSubcoreMesh(core_axis_name="c", subcore_axis_name="s"),
    compiler_params=pltpu.CompilerParams(use_tc_tiling_on_sc=False),
    scratch_types=[
        pltpu.VMEM((_NCHUNK, _CHUNK), jnp.int32),
        pltpu.VMEM((_CHUNK, _D), jnp.float32),
        pltpu.VMEM((_CHUNK, _D), jnp.float32),
        pltpu.VMEM((_CHUNK, _D), jnp.float32),
        pltpu.VMEM((_CHUNK, _D), jnp.float32),
        pltpu.SemaphoreType.DMA,
        pltpu.SemaphoreType.DMA,
        pltpu.SemaphoreType.DMA,
        pltpu.SemaphoreType.DMA,
    ],
)
def _sc_gather(table_hbm, src_hbm, out_hbm, *scratch):
    _sc_body(table_hbm, src_hbm, out_hbm, *scratch)


def _fin_body(rows_ref, out_ref):
    xt = rows_ref[...].T
    out_ref[...] = jnp.concatenate(
        [xt[0:_D, :], xt[_D:2 * _D, :]], axis=1).reshape(1, 1, _D, _B)


def _fin_body2(rows_ref, prev_ref, out_ref):
    del prev_ref
    _fin_body(rows_ref, out_ref)


def _fin_first(rows128_e):
    return pl.pallas_call(
        _fin_body,
        grid=(_L,),
        in_specs=[
            pl.BlockSpec((_B // 2, 2 * _D), lambda l: (l, 0)),
        ],
        out_specs=pl.BlockSpec((1, 1, _D, _B), lambda l: (1, l, 0, 0)),
        out_shape=jax.ShapeDtypeStruct((2, _L, _D, _B), jnp.float32),
    )(rows128_e)


def _fin_second(rows128_t, prev):
    return pl.pallas_call(
        _fin_body2,
        grid=(_L,),
        in_specs=[
            pl.BlockSpec((_B // 2, 2 * _D), lambda l: (l, 0)),
            pl.BlockSpec(memory_space=pl.ANY),
        ],
        out_specs=pl.BlockSpec((1, 1, _D, _B), lambda l: (0, l, 0, 0)),
        out_shape=jax.ShapeDtypeStruct((2, _L, _D, _B), jnp.float32),
        input_output_aliases={1: 0},
    )(rows128_t, prev)


def kernel(titles, entities, word_table, entity_table, W, b):
    transformed = _transform_table(entity_table.T, W, b).reshape(2 * _M_E, _D)
    word_rm = _transpose_word(word_table.T).reshape(2 * _M_W, _D)

    def _perm(x):
        return (x.T.reshape(_L, 2, _B // 2).swapaxes(1, 2)
                .reshape(-1).astype(jnp.int32))

    t = _perm(titles)
    e = _perm(entities)
    src_t = jnp.where(t < _M_W, 2 * t, 2 * (t - _SB_W * _BLK) + 1)
    src_e = jnp.where(e < _M_E, 2 * e, 2 * (e - _SB_E * _BLK) + 1)

    shape3 = (_NW, _NCHUNK, _CHUNK)
    rows_e = _sc_gather(transformed, src_e.reshape(shape3))
    rows_w = _sc_gather(word_rm, src_t.reshape(shape3))
    out_half = _fin_first(rows_e.reshape(_POS // 2, 2 * _D))
    out_p = _fin_second(rows_w.reshape(_POS // 2, 2 * _D), out_half)
    return jnp.transpose(out_p, (3, 0, 1, 2))


# speedup vs baseline: 1.2568x; 1.2568x over previous
import functools

import jax
import jax.numpy as jnp
from jax import lax
from jax.experimental import pallas as pl
from jax.experimental.pallas import tpu as pltpu
from jax.experimental.pallas import tpu_sc as plsc

_VOCAB = 1000000
_D = 64
_ENT_V = 100000
_ENT_D = 100
_B = 16384
_L = 20

_NC = 2
_NS = 16
_NW = _NC * _NS
_POS = _B * _L
_PER_W = _POS // _NW
_CHUNK = 128
_NCHUNK = _PER_W // _CHUNK
_NBUF = 4

_BLK = 4096
_M_W = 503808
_SB_W = 122
_M_E = 53248
_SB_E = 12
_FIN_NB = 512


def _transform_body(xa_ref, xb_ref, w_ref, b_ref, out_ref):
    dn = (((0,), (0,)), ((), ()))
    acc_a = lax.dot_general(xa_ref[...], w_ref[...], dn,
                            preferred_element_type=jnp.float32)
    acc_b = lax.dot_general(xb_ref[...], w_ref[...], dn,
                            preferred_element_type=jnp.float32)
    bias = b_ref[...]
    out_ref[...] = jnp.tanh(
        jnp.concatenate([acc_a + bias, acc_b + bias], axis=1))


def _transform_table(entity_table_t, W, b):
    nblk = _M_E // _BLK
    return pl.pallas_call(
        _transform_body,
        grid=(nblk,),
        in_specs=[
            pl.BlockSpec((_ENT_D, _BLK), lambda i: (0, i)),
            pl.BlockSpec((_ENT_D, _BLK), lambda i: (0, _SB_E + i)),
            pl.BlockSpec((_ENT_D, _D), lambda i: (0, 0)),
            pl.BlockSpec((1, _D), lambda i: (0, 0)),
        ],
        out_specs=pl.BlockSpec((_BLK, 2 * _D), lambda i: (i, 0)),
        out_shape=jax.ShapeDtypeStruct((_M_E, 2 * _D), jnp.float32),
    )(entity_table_t, entity_table_t, W, b.reshape(1, _D))


def _wt_body(xa_ref, xb_ref, out_ref):
    i0 = lax.broadcasted_iota(jnp.int32, (_D, _D), 0)
    i1 = lax.broadcasted_iota(jnp.int32, (_D, _D), 1)
    eye = jnp.where(i0 == i1, 1.0, 0.0).astype(jnp.float32)
    dn = (((0,), (0,)), ((), ()))
    ya = lax.dot_general(xa_ref[...], eye, dn,
                         preferred_element_type=jnp.float32)
    yb = lax.dot_general(xb_ref[...], eye, dn,
                         preferred_element_type=jnp.float32)
    out_ref[...] = jnp.concatenate([ya, yb], axis=1)


def _transpose_word(word_table_t):
    nblk = _M_W // _BLK
    return pl.pallas_call(
        _wt_body,
        grid=(nblk,),
        in_specs=[
            pl.BlockSpec((_D, _BLK), lambda i: (0, i)),
            pl.BlockSpec((_D, _BLK), lambda i: (0, _SB_W + i)),
        ],
        out_specs=pl.BlockSpec((_BLK, 2 * _D), lambda i: (i, 0)),
        out_shape=jax.ShapeDtypeStruct((_M_W, 2 * _D), jnp.float32),
    )(word_table_t, word_table_t)


def _sc_body(table_hbm, src_hbm, out_hbm, src_v,
             buf0, buf1, buf2, buf3, sem0, sem1, sem2, sem3):
    c = lax.axis_index("c")
    s = lax.axis_index("s")
    wid = s * _NC + c
    bufs = (buf0, buf1, buf2, buf3)
    sems = (sem0, sem1, sem2, sem3)
    row0 = wid * _PER_W

    pltpu.sync_copy(src_hbm.at[wid], src_v)

    for t in range(_NBUF):
        pltpu.async_copy(table_hbm.at[src_v.at[t]], bufs[t], sems[t])

    def outer(o, carry):
        for t in range(_NBUF):
            j = o * _NBUF + t
            pltpu.make_async_copy(
                table_hbm.at[src_v.at[j]], bufs[t], sems[t]).wait()
            pltpu.async_copy(
                bufs[t], out_hbm.at[pl.ds(row0 + j * _CHUNK, _CHUNK)],
                sems[t]).wait()
            pltpu.async_copy(
                table_hbm.at[src_v.at[j + _NBUF]], bufs[t], sems[t])
        return carry

    lax.fori_loop(0, _NCHUNK // _NBUF - 1, outer, 0)

    for t in range(_NBUF):
        j = (_NCHUNK - _NBUF) + t
        pltpu.make_async_copy(
            table_hbm.at[src_v.at[j]], bufs[t], sems[t]).wait()
        pltpu.async_copy(
            bufs[t], out_hbm.at[pl.ds(row0 + j * _CHUNK, _CHUNK)],
            sems[t]).wait()


@functools.partial(
    pl.kernel,
    out_type=jax.ShapeDtypeStruct((_POS, _D), jnp.float32),
    mesh=plsc.VectorSubcoreMesh(core_axis_name="c", subcore_axis_name="s"),
    compiler_params=pltpu.CompilerParams(use_tc_tiling_on_sc=False),
    scratch_types=[
        pltpu.VMEM((_NCHUNK, _CHUNK), jnp.int32),
        pltpu.VMEM((_CHUNK, _D), jnp.float32),
        pltpu.VMEM((_CHUNK, _D), jnp.float32),
        pltpu.VMEM((_CHUNK, _D), jnp.float32),
        pltpu.VMEM((_CHUNK, _D), jnp.float32),
        pltpu.SemaphoreType.DMA,
        pltpu.SemaphoreType.DMA,
        pltpu.SemaphoreType.DMA,
        pltpu.SemaphoreType.DMA,
    ],
)
def _sc_gather(table_hbm, src_hbm, out_hbm, *scratch):
    _sc_body(table_hbm, src_hbm, out_hbm, *scratch)


def _fin_body(rows_ref, out_ref):
    xt = rows_ref[...].T
    out_ref[...] = jnp.concatenate(
        [xt[0:_D, :], xt[_D:2 * _D, :]], axis=1).reshape(1, 1, _D, _B)


def _fin_body2(rows_ref, prev_ref, out_ref):
    del prev_ref
    _fin_body(rows_ref, out_ref)


def _fin_first(rows128_e):
    return pl.pallas_call(
        _fin_body,
        grid=(_L,),
        in_specs=[
            pl.BlockSpec((_B // 2, 2 * _D), lambda l: (l, 0)),
        ],
        out_specs=pl.BlockSpec((1, 1, _D, _B), lambda l: (1, l, 0, 0)),
        out_shape=jax.ShapeDtypeStruct((2, _L, _D, _B), jnp.float32),
    )(rows128_e)


def _fin_second(rows128_t, prev):
    return pl.pallas_call(
        _fin_body2,
        grid=(_L,),
        in_specs=[
            pl.BlockSpec((_B // 2, 2 * _D), lambda l: (l, 0)),
            pl.BlockSpec(memory_space=pl.ANY),
        ],
        out_specs=pl.BlockSpec((1, 1, _D, _B), lambda l: (0, l, 0, 0)),
        out_shape=jax.ShapeDtypeStruct((2, _L, _D, _B), jnp.float32),
        input_output_aliases={1: 0},
    )(rows128_t, prev)


def kernel(titles, entities, word_table, entity_table, W, b):
    transformed = _transform_table(entity_table.T, W, b).reshape(2 * _M_E, _D)
    word_rm = _transpose_word(word_table.T).reshape(2 * _M_W, _D)

    order = jnp.stack(
        [jnp.arange(_B // 2, dtype=jnp.int32),
         jnp.arange(_B // 2, dtype=jnp.int32) + _B // 2],
        axis=1).reshape(-1)
    t = titles.T[:, order].reshape(-1).astype(jnp.int32)
    e = entities.T[:, order].reshape(-1).astype(jnp.int32)
    src_t = jnp.where(t < _M_W, 2 * t, 2 * (t - _SB_W * _BLK) + 1)
    src_e = jnp.where(e < _M_E, 2 * e, 2 * (e - _SB_E * _BLK) + 1)

    shape3 = (_NW, _NCHUNK, _CHUNK)
    rows_e = _sc_gather(transformed, src_e.reshape(shape3))
    rows_w = _sc_gather(word_rm, src_t.reshape(shape3))
    out_half = _fin_first(rows_e.reshape(_POS // 2, 2 * _D))
    out_p = _fin_second(rows_w.reshape(_POS // 2, 2 * _D), out_half)
    return jnp.transpose(out_p, (3, 0, 1, 2))
